# baseline (speedup 1.0000x reference)
import functools

import jax
import jax.numpy as jnp
from jax import lax
from jax.experimental import pallas as pl
from jax.experimental.pallas import tpu as pltpu
from jax.experimental.pallas import tpu_sc as plsc

N = 10000
E = 320000
D = 128
DE = 16

NP_ = 10112
CH = 96
NTILES = 32
CPT = 106
CPT0 = 68
CPT1 = 144
EPT = CPT * CH
EP = NTILES * EPT
NCH = EP // CH
EPP = 327680
RPS = NP_ // 16

_mesh = plsc.VectorSubcoreMesh(
    core_axis_name="c", subcore_axis_name="s", num_cores=2, num_subcores=16
)


def _sc_body(h_hbm, p_hbm, src_hbm, dst_hbm, out_hbm, acc,
             sb0, sb1, db0, db1, g0, g1, p0, p1,
             semi0, semi1, semid0, semid1,
             semg0, semg1, semp0, semp1, semsc0, semsc1):
    c = lax.axis_index("c")
    s = lax.axis_index("s")
    wid = s * 2 + c
    cpt = jnp.where(c == 0, CPT0, CPT1)
    cbase = jnp.where(c == 0, s * CPT0, 16 * CPT0 + s * CPT1)

    zeros16 = jnp.zeros((16,), jnp.float32)

    @pl.loop(0, CH)
    def _(r):
        for cc in range(8):
            g0[r, pl.ds(cc * 16, 16)] = zeros16

    for t in range(RPS // CH):
        pltpu.sync_copy(g0, acc.at[pl.ds(s * RPS + t * CH, CH)])
    pltpu.sync_copy(g0.at[pl.ds(0, RPS % CH)],
                    acc.at[pl.ds(s * RPS + (RPS // CH) * CH, RPS % CH)])
    plsc.subcore_barrier()

    pltpu.async_copy(src_hbm.at[pl.ds(cbase, 1)], sb0, semi0)
    pltpu.async_copy(src_hbm.at[pl.ds(cbase + 1, 1)], sb1, semi1)
    pltpu.async_copy(dst_hbm.at[pl.ds(cbase, 1)], db0, semid0)
    pltpu.make_async_copy(src_hbm.at[pl.ds(cbase, 1)], sb0, semi0).wait()
    pltpu.async_copy(h_hbm.at[sb0.at[0]], g0, semg0)
    pltpu.async_copy(p_hbm.at[pl.ds(cbase * CH, CH)], p0, semp0)

    def step(j, sb_s, db_s, g_s, p_s, semi_s, semid_s, semg_s, semp_s,
             semsc_s, sb_o, db_o, g_o, p_o, semi_o, semid_o, semg_o,
             semp_o, semsc_o):
        pltpu.make_async_copy(h_hbm.at[sb_s.at[0]], g_s, semg_s).wait()
        j2 = jnp.minimum(j + 2, cpt - 1)
        pltpu.async_copy(src_hbm.at[pl.ds(cbase + j2, 1)], sb_s, semi_s)

        @pl.when(j > 0)
        def _():
            pltpu.make_async_copy(g_o, acc.at[db_o.at[0]], semsc_o).wait()
        j1 = jnp.minimum(j + 1, cpt - 1)
        pltpu.async_copy(dst_hbm.at[pl.ds(cbase + j1, 1)], db_o, semid_o)

        pltpu.make_async_copy(src_hbm.at[pl.ds(0, 1)], sb_o, semi_o).wait()
        pltpu.async_copy(h_hbm.at[sb_o.at[0]], g_o, semg_o)
        pltpu.async_copy(p_hbm.at[pl.ds((cbase + j1) * CH, CH)], p_o,
                         semp_o)

        pltpu.make_async_copy(p_hbm.at[pl.ds(0, CH)], p_s, semp_s).wait()
        @pl.loop(0, CH)
        def _(r):
            for cc in range(8):
                sl = pl.ds(cc * 16, 16)
                g_s[r, sl] = jnp.maximum(g_s[r, sl] + p_s[r, sl], 0.0)

        pltpu.make_async_copy(dst_hbm.at[pl.ds(0, 1)], db_s, semid_s).wait()
        pltpu.async_copy(g_s, acc.at[db_s.at[0]], semsc_s, add=True)

    @pl.loop(0, cpt // 2)
    def _(i):
        step(2 * i,
             sb0, db0, g0, p0, semi0, semid0, semg0, semp0, semsc0,
             sb1, db1, g1, p1, semi1, semid1, semg1, semp1, semsc1)
        step(2 * i + 1,
             sb1, db1, g1, p1, semi1, semid1, semg1, semp1, semsc1,
             sb0, db0, g0, p0, semi0, semid0, semg0, semp0, semsc0)

    pltpu.make_async_copy(h_hbm.at[sb0.at[0]], g0, semg0).wait()
    pltpu.make_async_copy(p_hbm.at[pl.ds(0, CH)], p0, semp0).wait()
    pltpu.make_async_copy(src_hbm.at[pl.ds(0, 1)], sb1, semi1).wait()
    pltpu.make_async_copy(dst_hbm.at[pl.ds(0, 1)], db0, semid0).wait()
    pltpu.make_async_copy(g1, acc.at[db1.at[0]], semsc1).wait()

    plsc.subcore_barrier()
    pltpu.sync_copy(acc.at[pl.ds(s * RPS, RPS)],
                    out_hbm.at[pl.ds(c * NP_ + s * RPS, RPS)])


_sc_aggregate = functools.partial(
    pl.kernel,
    out_type=jax.ShapeDtypeStruct((2 * NP_, D), jnp.float32),
    mesh=_mesh,
    scratch_types=[
        pltpu.VMEM_SHARED((NP_, D), jnp.float32),
        pltpu.VMEM((1, CH), jnp.int32),
        pltpu.VMEM((1, CH), jnp.int32),
        pltpu.VMEM((1, CH), jnp.int32),
        pltpu.VMEM((1, CH), jnp.int32),
        pltpu.VMEM((CH, D), jnp.float32),
        pltpu.VMEM((CH, D), jnp.float32),
        pltpu.VMEM((CH, D), jnp.float32),
        pltpu.VMEM((CH, D), jnp.float32),
        pltpu.SemaphoreType.DMA,
        pltpu.SemaphoreType.DMA,
        pltpu.SemaphoreType.DMA,
        pltpu.SemaphoreType.DMA,
        pltpu.SemaphoreType.DMA,
        pltpu.SemaphoreType.DMA,
        pltpu.SemaphoreType.DMA,
        pltpu.SemaphoreType.DMA,
        pltpu.SemaphoreType.DMA,
        pltpu.SemaphoreType.DMA,
    ],
)(_sc_body)


def _mm_body(x_ref, w_ref, o_ref):
    o_ref[...] = jax.lax.dot_general(
        x_ref[...], w_ref[...], (((1,), (0,)), ((), ())),
        preferred_element_type=jnp.float32)


def _mm_bias_body(x_ref, w_ref, b_ref, o_ref):
    o_ref[...] = jax.lax.dot_general(
        x_ref[...], w_ref[...], (((1,), (0,)), ((), ())),
        preferred_element_type=jnp.float32) + b_ref[...]


def _final_body(x_ref, a0_ref, a1_ref, wa_ref, wb_ref, b_ref, o_ref):
    agg = a0_ref[0] + a1_ref[0]
    acc = jax.lax.dot_general(
        x_ref[...], wa_ref[...], (((1,), (0,)), ((), ())),
        preferred_element_type=jnp.float32)
    acc += jax.lax.dot_general(
        agg, wb_ref[...], (((1,), (0,)), ((), ())),
        preferred_element_type=jnp.float32)
    o_ref[...] = jnp.maximum(acc + b_ref[...], 0.0)


def _tc_matmul(x, w, block_rows):
    m = x.shape[0]
    return pl.pallas_call(
        _mm_body,
        grid=(m // block_rows,),
        in_specs=[
            pl.BlockSpec((block_rows, x.shape[1]), lambda i: (i, 0)),
            pl.BlockSpec((w.shape[0], w.shape[1]), lambda i: (0, 0)),
        ],
        out_specs=pl.BlockSpec((block_rows, w.shape[1]), lambda i: (i, 0)),
        out_shape=jax.ShapeDtypeStruct((m, w.shape[1]), jnp.float32),
    )(x, w)


def _tc_p_matmul(ea, w, b, block_rows):
    last = E // block_rows - 1
    return pl.pallas_call(
        _mm_bias_body,
        grid=(EPP // block_rows,),
        in_specs=[
            pl.BlockSpec((block_rows, DE), lambda i: (jnp.minimum(i, last), 0)),
            pl.BlockSpec((DE, D), lambda i: (0, 0)),
            pl.BlockSpec((1, D), lambda i: (0, 0)),
        ],
        out_specs=pl.BlockSpec((block_rows, D), lambda i: (i, 0)),
        out_shape=jax.ShapeDtypeStruct((EPP, D), jnp.float32),
    )(ea, w, b)


def _tc_final(x_ap, partials3, wa, wb, b, block_rows):
    return pl.pallas_call(
        _final_body,
        grid=(N // block_rows,),
        in_specs=[
            pl.BlockSpec((block_rows, D), lambda i: (i, 0)),
            pl.BlockSpec((1, block_rows, D), lambda i: (0, i, 0)),
            pl.BlockSpec((1, block_rows, D), lambda i: (1, i, 0)),
            pl.BlockSpec((D, D), lambda i: (0, 0)),
            pl.BlockSpec((D, D), lambda i: (0, 0)),
            pl.BlockSpec((1, D), lambda i: (0, 0)),
        ],
        out_specs=pl.BlockSpec((block_rows, D), lambda i: (i, 0)),
        out_shape=jax.ShapeDtypeStruct((N, D), jnp.float32),
    )(x_ap, partials3, partials3, wa, wb, b)


def kernel(x_ue, x_ap, edge_index, edge_attr, W1, b1, W2, b2):
    src = edge_index[0].astype(jnp.int32)
    dst = edge_index[1].astype(jnp.int32)

    pad_e = EP - E
    src_p = jnp.concatenate([src, jnp.zeros((pad_e,), jnp.int32)])
    pad_dst = N + (jnp.arange(pad_e, dtype=jnp.int32) % (NP_ - N))
    dst_p = jnp.concatenate([dst, pad_dst])
    src2 = src_p.reshape(NCH, CH)
    dst2 = dst_p.reshape(NCH, CH)

    W1a, W1b = W1[:DE], W1[DE:]
    W2a, W2b = W2[:D], W2[D:]

    H = _tc_matmul(x_ue, W1b, 400)
    P = _tc_p_matmul(edge_attr, W1a, b1.reshape(1, D), 2560)
    partials = _sc_aggregate(H, P, src2, dst2)
    partials3 = partials.reshape(2, NP_, D)
    return _tc_final(x_ap, partials3, W2a, W2b, b2.reshape(1, D), 400)


# speedup vs baseline: 1.1270x; 1.1270x over previous
import functools

import jax
import jax.numpy as jnp
from jax import lax
from jax.experimental import pallas as pl
from jax.experimental.pallas import tpu as pltpu
from jax.experimental.pallas import tpu_sc as plsc

N = 10000
E = 320000
D = 128
DE = 16

NP_ = 10112
CH = 96
NTILES = 32
CPT = 106
CPT0 = 144
CPT1 = 68
EPT = CPT * CH
EP = NTILES * EPT
NCH = EP // CH
EPP = 327680
RPS = NP_ // 16

_mesh = plsc.VectorSubcoreMesh(
    core_axis_name="c", subcore_axis_name="s", num_cores=2, num_subcores=16
)


def _sc_body(h_hbm, p_hbm, src_hbm, dst_hbm, out_hbm, acc,
             sb0, sb1, db0, db1, g0, g1, p0, p1,
             semi0, semi1, semid0, semid1,
             semg0, semg1, semp0, semp1, semsc0, semsc1):
    c = lax.axis_index("c")
    s = lax.axis_index("s")
    wid = s * 2 + c
    cpt = jnp.where(c == 0, CPT0, CPT1)
    cbase = jnp.where(c == 0, s * CPT0, 16 * CPT0 + s * CPT1)

    zeros16 = jnp.zeros((16,), jnp.float32)

    @pl.loop(0, CH)
    def _(r):
        for cc in range(8):
            g0[r, pl.ds(cc * 16, 16)] = zeros16

    for t in range(RPS // CH):
        pltpu.sync_copy(g0, acc.at[pl.ds(s * RPS + t * CH, CH)])
    pltpu.sync_copy(g0.at[pl.ds(0, RPS % CH)],
                    acc.at[pl.ds(s * RPS + (RPS // CH) * CH, RPS % CH)])
    plsc.subcore_barrier()

    pltpu.async_copy(src_hbm.at[pl.ds(cbase, 1)], sb0, semi0)
    pltpu.async_copy(src_hbm.at[pl.ds(cbase + 1, 1)], sb1, semi1)
    pltpu.async_copy(dst_hbm.at[pl.ds(cbase, 1)], db0, semid0)
    pltpu.make_async_copy(src_hbm.at[pl.ds(cbase, 1)], sb0, semi0).wait()
    pltpu.async_copy(h_hbm.at[sb0.at[0]], g0, semg0)
    pltpu.async_copy(p_hbm.at[pl.ds(cbase * CH, CH)], p0, semp0)

    def step(j, sb_s, db_s, g_s, p_s, semi_s, semid_s, semg_s, semp_s,
             semsc_s, sb_o, db_o, g_o, p_o, semi_o, semid_o, semg_o,
             semp_o, semsc_o):
        pltpu.make_async_copy(h_hbm.at[sb_s.at[0]], g_s, semg_s).wait()
        j2 = jnp.minimum(j + 2, cpt - 1)
        pltpu.async_copy(src_hbm.at[pl.ds(cbase + j2, 1)], sb_s, semi_s)

        @pl.when(j > 0)
        def _():
            pltpu.make_async_copy(g_o, acc.at[db_o.at[0]], semsc_o).wait()
        j1 = jnp.minimum(j + 1, cpt - 1)
        pltpu.async_copy(dst_hbm.at[pl.ds(cbase + j1, 1)], db_o, semid_o)

        pltpu.make_async_copy(src_hbm.at[pl.ds(0, 1)], sb_o, semi_o).wait()
        pltpu.async_copy(h_hbm.at[sb_o.at[0]], g_o, semg_o)
        pltpu.async_copy(p_hbm.at[pl.ds((cbase + j1) * CH, CH)], p_o,
                         semp_o)

        pltpu.make_async_copy(p_hbm.at[pl.ds(0, CH)], p_s, semp_s).wait()
        @pl.loop(0, CH)
        def _(r):
            for cc in range(8):
                sl = pl.ds(cc * 16, 16)
                g_s[r, sl] = jnp.maximum(g_s[r, sl] + p_s[r, sl], 0.0)

        pltpu.make_async_copy(dst_hbm.at[pl.ds(0, 1)], db_s, semid_s).wait()
        pltpu.async_copy(g_s, acc.at[db_s.at[0]], semsc_s, add=True)

    @pl.loop(0, cpt // 2)
    def _(i):
        step(2 * i,
             sb0, db0, g0, p0, semi0, semid0, semg0, semp0, semsc0,
             sb1, db1, g1, p1, semi1, semid1, semg1, semp1, semsc1)
        step(2 * i + 1,
             sb1, db1, g1, p1, semi1, semid1, semg1, semp1, semsc1,
             sb0, db0, g0, p0, semi0, semid0, semg0, semp0, semsc0)

    pltpu.make_async_copy(h_hbm.at[sb0.at[0]], g0, semg0).wait()
    pltpu.make_async_copy(p_hbm.at[pl.ds(0, CH)], p0, semp0).wait()
    pltpu.make_async_copy(src_hbm.at[pl.ds(0, 1)], sb1, semi1).wait()
    pltpu.make_async_copy(dst_hbm.at[pl.ds(0, 1)], db0, semid0).wait()
    pltpu.make_async_copy(g1, acc.at[db1.at[0]], semsc1).wait()

    plsc.subcore_barrier()
    pltpu.sync_copy(acc.at[pl.ds(s * RPS, RPS)],
                    out_hbm.at[pl.ds(c * NP_ + s * RPS, RPS)])


_sc_aggregate = functools.partial(
    pl.kernel,
    out_type=jax.ShapeDtypeStruct((2 * NP_, D), jnp.float32),
    mesh=_mesh,
    scratch_types=[
        pltpu.VMEM_SHARED((NP_, D), jnp.float32),
        pltpu.VMEM((1, CH), jnp.int32),
        pltpu.VMEM((1, CH), jnp.int32),
        pltpu.VMEM((1, CH), jnp.int32),
        pltpu.VMEM((1, CH), jnp.int32),
        pltpu.VMEM((CH, D), jnp.float32),
        pltpu.VMEM((CH, D), jnp.float32),
        pltpu.VMEM((CH, D), jnp.float32),
        pltpu.VMEM((CH, D), jnp.float32),
        pltpu.SemaphoreType.DMA,
        pltpu.SemaphoreType.DMA,
        pltpu.SemaphoreType.DMA,
        pltpu.SemaphoreType.DMA,
        pltpu.SemaphoreType.DMA,
        pltpu.SemaphoreType.DMA,
        pltpu.SemaphoreType.DMA,
        pltpu.SemaphoreType.DMA,
        pltpu.SemaphoreType.DMA,
        pltpu.SemaphoreType.DMA,
    ],
)(_sc_body)


def _mm_body(x_ref, w_ref, o_ref):
    o_ref[...] = jax.lax.dot_general(
        x_ref[...], w_ref[...], (((1,), (0,)), ((), ())),
        preferred_element_type=jnp.float32)


def _mm_bias_body(x_ref, w_ref, b_ref, o_ref):
    o_ref[...] = jax.lax.dot_general(
        x_ref[...], w_ref[...], (((1,), (0,)), ((), ())),
        preferred_element_type=jnp.float32) + b_ref[...]


def _final_body(x_ref, a0_ref, a1_ref, wa_ref, wb_ref, b_ref, o_ref):
    agg = a0_ref[0] + a1_ref[0]
    acc = jax.lax.dot_general(
        x_ref[...], wa_ref[...], (((1,), (0,)), ((), ())),
        preferred_element_type=jnp.float32)
    acc += jax.lax.dot_general(
        agg, wb_ref[...], (((1,), (0,)), ((), ())),
        preferred_element_type=jnp.float32)
    o_ref[...] = jnp.maximum(acc + b_ref[...], 0.0)


def _tc_matmul(x, w, block_rows):
    m = x.shape[0]
    return pl.pallas_call(
        _mm_body,
        grid=(m // block_rows,),
        in_specs=[
            pl.BlockSpec((block_rows, x.shape[1]), lambda i: (i, 0)),
            pl.BlockSpec((w.shape[0], w.shape[1]), lambda i: (0, 0)),
        ],
        out_specs=pl.BlockSpec((block_rows, w.shape[1]), lambda i: (i, 0)),
        out_shape=jax.ShapeDtypeStruct((m, w.shape[1]), jnp.float32),
    )(x, w)


def _tc_p_matmul(ea, w, b, block_rows):
    last = E // block_rows - 1
    return pl.pallas_call(
        _mm_bias_body,
        grid=(EPP // block_rows,),
        in_specs=[
            pl.BlockSpec((block_rows, DE), lambda i: (jnp.minimum(i, last), 0)),
            pl.BlockSpec((DE, D), lambda i: (0, 0)),
            pl.BlockSpec((1, D), lambda i: (0, 0)),
        ],
        out_specs=pl.BlockSpec((block_rows, D), lambda i: (i, 0)),
        out_shape=jax.ShapeDtypeStruct((EPP, D), jnp.float32),
    )(ea, w, b)


def _tc_final(x_ap, partials3, wa, wb, b, block_rows):
    return pl.pallas_call(
        _final_body,
        grid=(N // block_rows,),
        in_specs=[
            pl.BlockSpec((block_rows, D), lambda i: (i, 0)),
            pl.BlockSpec((1, block_rows, D), lambda i: (0, i, 0)),
            pl.BlockSpec((1, block_rows, D), lambda i: (1, i, 0)),
            pl.BlockSpec((D, D), lambda i: (0, 0)),
            pl.BlockSpec((D, D), lambda i: (0, 0)),
            pl.BlockSpec((1, D), lambda i: (0, 0)),
        ],
        out_specs=pl.BlockSpec((block_rows, D), lambda i: (i, 0)),
        out_shape=jax.ShapeDtypeStruct((N, D), jnp.float32),
    )(x_ap, partials3, partials3, wa, wb, b)


def kernel(x_ue, x_ap, edge_index, edge_attr, W1, b1, W2, b2):
    src = edge_index[0].astype(jnp.int32)
    dst = edge_index[1].astype(jnp.int32)

    pad_e = EP - E
    src_p = jnp.concatenate([src, jnp.zeros((pad_e,), jnp.int32)])
    pad_dst = N + (jnp.arange(pad_e, dtype=jnp.int32) % (NP_ - N))
    dst_p = jnp.concatenate([dst, pad_dst])
    src2 = src_p.reshape(NCH, CH)
    dst2 = dst_p.reshape(NCH, CH)

    W1a, W1b = W1[:DE], W1[DE:]
    W2a, W2b = W2[:D], W2[D:]

    H = _tc_matmul(x_ue, W1b, 400)
    P = _tc_p_matmul(edge_attr, W1a, b1.reshape(1, D), 2560)
    partials = _sc_aggregate(H, P, src2, dst2)
    partials3 = partials.reshape(2, NP_, D)
    return _tc_final(x_ap, partials3, W2a, W2b, b2.reshape(1, D), 400)
